# baseline (speedup 1.0000x reference)
import functools

import jax
import jax.numpy as jnp
from jax import lax
from jax.experimental import pallas as pl
from jax.experimental.pallas import tpu as pltpu
from jax.experimental.pallas import tpu_sc as plsc

N_NODES = 100000
FT = 128
NB_REL = 4
DEG = 32
BATCH = 4096
SEGS = BATCH * NB_REL

NC = 2
NS = 16
NW = NC * NS
SEG_PER_W = SEGS // NW
SEGS_PER_CHUNK = 4
ROWS_PER_CHUNK = SEGS_PER_CHUNK * DEG
CHUNKS = SEG_PER_W // SEGS_PER_CHUNK
LANES = 16
VPR = FT // LANES


NBUF = 6


def _seg_mean_body(idx_hbm, feat_hbm, out_hbm, idx_v, rows_v, out_v,
                   gs0, gs1, gs2, gs3, gs4, gs5, os0, os1):
    gsems = (gs0, gs1, gs2, gs3, gs4, gs5)
    osems = (os0, os1)
    wid = lax.axis_index("s") * NC + lax.axis_index("c")
    seg_base = wid * SEG_PER_W

    pltpu.sync_copy(idx_hbm.at[pl.ds(seg_base * DEG, SEG_PER_W * DEG)], idx_v)

    def gather(c, b):
        idx_slice = idx_v.at[pl.ds(c * ROWS_PER_CHUNK, ROWS_PER_CHUNK)]
        return pltpu.make_async_copy(feat_hbm.at[idx_slice], rows_v.at[b],
                                     gsems[b])

    def out_copy(c, p):
        dst = out_hbm.at[pl.ds(seg_base + c * SEGS_PER_CHUNK,
                               SEGS_PER_CHUNK)]
        return pltpu.make_async_copy(out_v.at[p], dst, osems[p])

    for b in range(NBUF - 1):
        gather(b, b).start()

    n_groups = -(-CHUNKS // NBUF)

    @pl.loop(0, n_groups * NBUF, step=NBUF)
    def _group(c0):
        for b in range(NBUF):
            c = c0 + b
            p = b % 2

            @pl.when(c < CHUNKS)
            def _():
                gather(c, b).wait()
                nxt = c + NBUF - 1

                @pl.when(nxt < CHUNKS)
                def _():
                    gather(nxt, (b + NBUF - 1) % NBUF).start()

                @pl.when(c >= 2)
                def _():
                    out_copy(c - 2, p).wait()

                for s in range(SEGS_PER_CHUNK):
                    def body(d, acc):
                        return tuple(
                            acc[l] + rows_v[b, s * DEG + d,
                                            pl.ds(l * LANES, LANES)]
                            for l in range(VPR)
                        )
                    acc = lax.fori_loop(
                        0, DEG, body,
                        tuple(jnp.zeros((LANES,), jnp.float32)
                              for _ in range(VPR)))
                    for l in range(VPR):
                        out_v[p, s, pl.ds(l * LANES, LANES)] = (
                            acc[l] * (1.0 / DEG))

                out_copy(c, p).start()

    for c in (CHUNKS - 2, CHUNKS - 1):
        out_copy(c, c % 2).wait()


_seg_mean = functools.partial(
    pl.kernel,
    out_type=jax.ShapeDtypeStruct((SEGS, FT), jnp.float32),
    mesh=plsc.VectorSubcoreMesh(
        core_axis_name="c", subcore_axis_name="s",
        num_cores=NC, num_subcores=NS),
    scratch_types=[
        pltpu.VMEM((SEG_PER_W * DEG,), jnp.int32),
        pltpu.VMEM((NBUF, ROWS_PER_CHUNK, FT), jnp.float32),
        pltpu.VMEM((2, SEGS_PER_CHUNK, FT), jnp.float32),
    ] + [pltpu.SemaphoreType.DMA] * (NBUF + 2),
)(_seg_mean_body)


def _gcn_body(x_ref, w_ref, b_ref, a_ref, o_ref):
    a = a_ref[0, 0]
    x = x_ref[...]
    acc = jnp.zeros((x.shape[0], FT), jnp.float32)
    for r in range(NB_REL):
        h = jnp.dot(x[:, r, :], w_ref[r], preferred_element_type=jnp.float32)
        h = h + b_ref[r][None, :]
        h = jnp.where(h > 0, h, a * h)
        acc = acc + h
    o_ref[...] = acc * (1.0 / NB_REL)


def _gcn(v_in, W1, b1, a11):
    BB = 1024
    return pl.pallas_call(
        _gcn_body,
        grid=(BATCH // BB,),
        in_specs=[
            pl.BlockSpec((BB, NB_REL, FT), lambda i: (i, 0, 0)),
            pl.BlockSpec((NB_REL, FT, FT), lambda i: (0, 0, 0)),
            pl.BlockSpec((NB_REL, FT), lambda i: (0, 0)),
            pl.BlockSpec(memory_space=pltpu.SMEM),
        ],
        out_specs=pl.BlockSpec((BB, FT), lambda i: (i, 0)),
        out_shape=jax.ShapeDtypeStruct((BATCH, FT), jnp.float32),
    )(v_in, W1, b1, a11)


def kernel(node_list, neighbor_idx, features, k, W1, b1, prelu_a):
    del node_list, k
    idx_flat = neighbor_idx.reshape(-1).astype(jnp.int32)
    v_in = _seg_mean(idx_flat, features)
    v_in = v_in.reshape(BATCH, NB_REL, FT)
    a11 = jnp.asarray(prelu_a, jnp.float32).reshape(1, 1)
    return _gcn(v_in, W1, b1, a11)


# speedup vs baseline: 1.0215x; 1.0215x over previous
import functools

import jax
import jax.numpy as jnp
from jax import lax
from jax.experimental import pallas as pl
from jax.experimental.pallas import tpu as pltpu
from jax.experimental.pallas import tpu_sc as plsc

N_NODES = 100000
FT = 128
NB_REL = 4
DEG = 32
BATCH = 4096
SEGS = BATCH * NB_REL

NC = 2
NS = 16
NW = NC * NS
SEG_PER_W = SEGS // NW
SEGS_PER_CHUNK = 4
ROWS_PER_CHUNK = SEGS_PER_CHUNK * DEG
CHUNKS = SEG_PER_W // SEGS_PER_CHUNK
LANES = 16
VPR = FT // LANES
NBUF = 4


def _seg_mean_body(idx_hbm, feat_hbm, out_hbm, idx_v, rows_v, out_v,
                   gs0, gs1, gs2, gs3, os0, os1):
    gsems = (gs0, gs1, gs2, gs3)
    osems = (os0, os1)
    wid = lax.axis_index("s") * NC + lax.axis_index("c")
    seg_base = wid * SEG_PER_W

    pltpu.sync_copy(idx_hbm.at[pl.ds(seg_base * DEG, SEG_PER_W * DEG)], idx_v)

    def gather(c, b):
        idx_slice = idx_v.at[pl.ds(c * ROWS_PER_CHUNK, ROWS_PER_CHUNK)]
        return pltpu.make_async_copy(feat_hbm.at[idx_slice], rows_v.at[b],
                                     gsems[b])

    def out_copy(c, p):
        dst = out_hbm.at[pl.ds(seg_base + c * SEGS_PER_CHUNK,
                               SEGS_PER_CHUNK)]
        return pltpu.make_async_copy(out_v.at[p], dst, osems[p])

    for b in range(NBUF - 1):
        gather(b, b).start()

    @pl.loop(0, CHUNKS, step=NBUF)
    def _group(c0):
        for b in range(NBUF):
            c = c0 + b
            p = b % 2
            gather(c, b).wait()
            nxt = c + NBUF - 1

            @pl.when(nxt < CHUNKS)
            def _():
                gather(nxt, (b + NBUF - 1) % NBUF).start()

            @pl.when(c >= 2)
            def _():
                out_copy(c - 2, p).wait()

            for s in range(SEGS_PER_CHUNK):
                def body(d, acc):
                    return tuple(
                        acc[l] + rows_v[b, s * DEG + d,
                                        pl.ds(l * LANES, LANES)]
                        for l in range(VPR)
                    )
                acc = lax.fori_loop(
                    0, DEG, body,
                    tuple(jnp.zeros((LANES,), jnp.float32)
                          for _ in range(VPR)))
                for l in range(VPR):
                    out_v[p, s, pl.ds(l * LANES, LANES)] = acc[l] * (1.0 / DEG)

            out_copy(c, p).start()

    for c in (CHUNKS - 2, CHUNKS - 1):
        out_copy(c, c % 2).wait()


_seg_mean = functools.partial(
    pl.kernel,
    out_type=jax.ShapeDtypeStruct((SEGS, FT), jnp.float32),
    mesh=plsc.VectorSubcoreMesh(
        core_axis_name="c", subcore_axis_name="s",
        num_cores=NC, num_subcores=NS),
    scratch_types=[
        pltpu.VMEM((SEG_PER_W * DEG,), jnp.int32),
        pltpu.VMEM((NBUF, ROWS_PER_CHUNK, FT), jnp.float32),
        pltpu.VMEM((2, SEGS_PER_CHUNK, FT), jnp.float32),
    ] + [pltpu.SemaphoreType.DMA] * (NBUF + 2),
)(_seg_mean_body)


def _gcn_body(x_ref, w_ref, b_ref, a_ref, o_ref):
    a = a_ref[0, 0]
    x = x_ref[...]
    acc = jnp.zeros((x.shape[0], FT), jnp.float32)
    for r in range(NB_REL):
        h = jnp.dot(x[:, r, :], w_ref[r], preferred_element_type=jnp.float32)
        h = h + b_ref[r][None, :]
        h = jnp.where(h > 0, h, a * h)
        acc = acc + h
    o_ref[...] = acc * (1.0 / NB_REL)


def _gcn(v_in, W1, b1, a11):
    BB = 1024
    return pl.pallas_call(
        _gcn_body,
        grid=(BATCH // BB,),
        in_specs=[
            pl.BlockSpec((BB, NB_REL, FT), lambda i: (i, 0, 0)),
            pl.BlockSpec((NB_REL, FT, FT), lambda i: (0, 0, 0)),
            pl.BlockSpec((NB_REL, FT), lambda i: (0, 0)),
            pl.BlockSpec(memory_space=pltpu.SMEM),
        ],
        out_specs=pl.BlockSpec((BB, FT), lambda i: (i, 0)),
        out_shape=jax.ShapeDtypeStruct((BATCH, FT), jnp.float32),
    )(v_in, W1, b1, a11)


def kernel(node_list, neighbor_idx, features, k, W1, b1, prelu_a):
    del node_list, k
    idx_flat = neighbor_idx.reshape(-1).astype(jnp.int32)
    v_in = _seg_mean(idx_flat, features)
    v_in = v_in.reshape(BATCH, NB_REL, FT)
    a11 = jnp.asarray(prelu_a, jnp.float32).reshape(1, 1)
    return _gcn(v_in, W1, b1, a11)


# speedup vs baseline: 1.0244x; 1.0029x over previous
import functools

import jax
import jax.numpy as jnp
from jax import lax
from jax.experimental import pallas as pl
from jax.experimental.pallas import tpu as pltpu
from jax.experimental.pallas import tpu_sc as plsc

N_NODES = 100000
FT = 128
NB_REL = 4
DEG = 32
BATCH = 4096
SEGS = BATCH * NB_REL

NC = 2
NS = 16
NW = NC * NS
SEG_PER_W = SEGS // NW
SEGS_PER_CHUNK = 2
ROWS_PER_CHUNK = SEGS_PER_CHUNK * DEG
CHUNKS = SEG_PER_W // SEGS_PER_CHUNK
LANES = 16
VPR = FT // LANES
NBUF = 8


def _seg_mean_body(idx_hbm, feat_hbm, out_hbm, idx_v, rows_v, out_v,
                   gs0, gs1, gs2, gs3, gs4, gs5, gs6, gs7, os0, os1):
    gsems = (gs0, gs1, gs2, gs3, gs4, gs5, gs6, gs7)
    osems = (os0, os1)
    wid = lax.axis_index("s") * NC + lax.axis_index("c")
    seg_base = wid * SEG_PER_W

    pltpu.sync_copy(idx_hbm.at[pl.ds(seg_base * DEG, SEG_PER_W * DEG)], idx_v)

    def gather(c, b):
        idx_slice = idx_v.at[pl.ds(c * ROWS_PER_CHUNK, ROWS_PER_CHUNK)]
        return pltpu.make_async_copy(feat_hbm.at[idx_slice], rows_v.at[b],
                                     gsems[b])

    def out_copy(c, p):
        dst = out_hbm.at[pl.ds(seg_base + c * SEGS_PER_CHUNK,
                               SEGS_PER_CHUNK)]
        return pltpu.make_async_copy(out_v.at[p], dst, osems[p])

    for b in range(NBUF - 1):
        gather(b, b).start()

    @pl.loop(0, CHUNKS, step=NBUF)
    def _group(c0):
        for b in range(NBUF):
            c = c0 + b
            p = b % 2
            gather(c, b).wait()
            nxt = c + NBUF - 1

            @pl.when(nxt < CHUNKS)
            def _():
                gather(nxt, (b + NBUF - 1) % NBUF).start()

            @pl.when(c >= 2)
            def _():
                out_copy(c - 2, p).wait()

            for s in range(SEGS_PER_CHUNK):
                def body(d, acc):
                    return tuple(
                        acc[l] + rows_v[b, s * DEG + d,
                                        pl.ds(l * LANES, LANES)]
                        for l in range(VPR)
                    )
                acc = lax.fori_loop(
                    0, DEG, body,
                    tuple(jnp.zeros((LANES,), jnp.float32)
                          for _ in range(VPR)))
                for l in range(VPR):
                    out_v[p, s, pl.ds(l * LANES, LANES)] = acc[l] * (1.0 / DEG)

            out_copy(c, p).start()

    for c in (CHUNKS - 2, CHUNKS - 1):
        out_copy(c, c % 2).wait()


_seg_mean = functools.partial(
    pl.kernel,
    out_type=jax.ShapeDtypeStruct((SEGS, FT), jnp.float32),
    mesh=plsc.VectorSubcoreMesh(
        core_axis_name="c", subcore_axis_name="s",
        num_cores=NC, num_subcores=NS),
    scratch_types=[
        pltpu.VMEM((SEG_PER_W * DEG,), jnp.int32),
        pltpu.VMEM((NBUF, ROWS_PER_CHUNK, FT), jnp.float32),
        pltpu.VMEM((2, SEGS_PER_CHUNK, FT), jnp.float32),
    ] + [pltpu.SemaphoreType.DMA] * (NBUF + 2),
)(_seg_mean_body)


def _gcn_body(x_ref, w_ref, b_ref, a_ref, o_ref):
    a = a_ref[0, 0]
    x = x_ref[...]
    acc = jnp.zeros((x.shape[0], FT), jnp.float32)
    for r in range(NB_REL):
        h = jnp.dot(x[:, r, :], w_ref[r], preferred_element_type=jnp.float32)
        h = h + b_ref[r][None, :]
        h = jnp.where(h > 0, h, a * h)
        acc = acc + h
    o_ref[...] = acc * (1.0 / NB_REL)


def _gcn(v_in, W1, b1, a11):
    BB = 1024
    return pl.pallas_call(
        _gcn_body,
        grid=(BATCH // BB,),
        in_specs=[
            pl.BlockSpec((BB, NB_REL, FT), lambda i: (i, 0, 0)),
            pl.BlockSpec((NB_REL, FT, FT), lambda i: (0, 0, 0)),
            pl.BlockSpec((NB_REL, FT), lambda i: (0, 0)),
            pl.BlockSpec(memory_space=pltpu.SMEM),
        ],
        out_specs=pl.BlockSpec((BB, FT), lambda i: (i, 0)),
        out_shape=jax.ShapeDtypeStruct((BATCH, FT), jnp.float32),
    )(v_in, W1, b1, a11)


def kernel(node_list, neighbor_idx, features, k, W1, b1, prelu_a):
    del node_list, k
    idx_flat = neighbor_idx.reshape(-1).astype(jnp.int32)
    v_in = _seg_mean(idx_flat, features)
    v_in = v_in.reshape(BATCH, NB_REL, FT)
    a11 = jnp.asarray(prelu_a, jnp.float32).reshape(1, 1)
    return _gcn(v_in, W1, b1, a11)


# speedup vs baseline: 1.0258x; 1.0014x over previous
import functools

import jax
import jax.numpy as jnp
from jax import lax
from jax.experimental import pallas as pl
from jax.experimental.pallas import tpu as pltpu
from jax.experimental.pallas import tpu_sc as plsc

N_NODES = 100000
FT = 128
NB_REL = 4
DEG = 32
BATCH = 4096
SEGS = BATCH * NB_REL

NC = 2
NS = 16
NW = NC * NS
SEG_PER_W = SEGS // NW
SEGS_PER_CHUNK = 2
ROWS_PER_CHUNK = SEGS_PER_CHUNK * DEG
CHUNKS = SEG_PER_W // SEGS_PER_CHUNK
LANES = 16
VPR = FT // LANES
NBUF = 8


def _seg_mean_body(idx_hbm, feat_hbm, out_hbm, idx_v, rows_v, out_v,
                   gs0, gs1, gs2, gs3, gs4, gs5, gs6, gs7, os0, os1):
    gsems = (gs0, gs1, gs2, gs3, gs4, gs5, gs6, gs7)
    osems = (os0, os1)
    wid = lax.axis_index("s") * NC + lax.axis_index("c")
    seg_base = wid * SEG_PER_W

    pltpu.sync_copy(idx_hbm.at[pl.ds(seg_base * DEG, SEG_PER_W * DEG)], idx_v)

    def gather(c, b):
        idx_slice = idx_v.at[pl.ds(c * ROWS_PER_CHUNK, ROWS_PER_CHUNK)]
        return pltpu.make_async_copy(feat_hbm.at[idx_slice], rows_v.at[b],
                                     gsems[b])

    def out_copy(c, p):
        dst = out_hbm.at[pl.ds(seg_base + c * SEGS_PER_CHUNK,
                               SEGS_PER_CHUNK)]
        return pltpu.make_async_copy(out_v.at[p], dst, osems[p])

    for b in range(NBUF - 1):
        gather(b, b).start()

    @pl.loop(0, CHUNKS, step=NBUF)
    def _group(c0):
        for b in range(NBUF):
            c = c0 + b
            p = b % 2
            gather(c, b).wait()
            nxt = c + NBUF - 1

            @pl.when(nxt < CHUNKS)
            def _():
                gather(nxt, (b + NBUF - 1) % NBUF).start()

            @pl.when(c >= 2)
            def _():
                out_copy(c - 2, p).wait()

            for s in range(SEGS_PER_CHUNK):
                def body(d, acc):
                    return tuple(
                        acc[l] + rows_v[b, s * DEG + d,
                                        pl.ds(l * LANES, LANES)]
                        for l in range(VPR)
                    )
                acc = lax.fori_loop(
                    0, DEG, body,
                    tuple(jnp.zeros((LANES,), jnp.float32)
                          for _ in range(VPR)))
                for l in range(VPR):
                    out_v[p, s, pl.ds(l * LANES, LANES)] = acc[l] * (1.0 / DEG)

            out_copy(c, p).start()

    for c in (CHUNKS - 2, CHUNKS - 1):
        out_copy(c, c % 2).wait()


_seg_mean = functools.partial(
    pl.kernel,
    out_type=jax.ShapeDtypeStruct((SEGS, FT), jnp.float32),
    mesh=plsc.VectorSubcoreMesh(
        core_axis_name="c", subcore_axis_name="s",
        num_cores=NC, num_subcores=NS),
    compiler_params=pltpu.CompilerParams(
        skip_device_barrier=True,
        disable_bounds_checks=True,
        disable_semaphore_checks=True,
    ),
    scratch_types=[
        pltpu.VMEM((SEG_PER_W * DEG,), jnp.int32),
        pltpu.VMEM((NBUF, ROWS_PER_CHUNK, FT), jnp.float32),
        pltpu.VMEM((2, SEGS_PER_CHUNK, FT), jnp.float32),
    ] + [pltpu.SemaphoreType.DMA] * (NBUF + 2),
)(_seg_mean_body)


def _gcn_body(x_ref, w_ref, b_ref, a_ref, o_ref):
    a = a_ref[0, 0]
    x = x_ref[...]
    acc = jnp.zeros((x.shape[0], FT), jnp.float32)
    for r in range(NB_REL):
        h = jnp.dot(x[:, r, :], w_ref[r], preferred_element_type=jnp.float32)
        h = h + b_ref[r][None, :]
        h = jnp.where(h > 0, h, a * h)
        acc = acc + h
    o_ref[...] = acc * (1.0 / NB_REL)


def _gcn(v_in, W1, b1, a11):
    BB = 1024
    return pl.pallas_call(
        _gcn_body,
        grid=(BATCH // BB,),
        in_specs=[
            pl.BlockSpec((BB, NB_REL, FT), lambda i: (i, 0, 0)),
            pl.BlockSpec((NB_REL, FT, FT), lambda i: (0, 0, 0)),
            pl.BlockSpec((NB_REL, FT), lambda i: (0, 0)),
            pl.BlockSpec(memory_space=pltpu.SMEM),
        ],
        out_specs=pl.BlockSpec((BB, FT), lambda i: (i, 0)),
        out_shape=jax.ShapeDtypeStruct((BATCH, FT), jnp.float32),
    )(v_in, W1, b1, a11)


def kernel(node_list, neighbor_idx, features, k, W1, b1, prelu_a):
    del node_list, k
    idx_flat = neighbor_idx.reshape(-1).astype(jnp.int32)
    v_in = _seg_mean(idx_flat, features)
    v_in = v_in.reshape(BATCH, NB_REL, FT)
    a11 = jnp.asarray(prelu_a, jnp.float32).reshape(1, 1)
    return _gcn(v_in, W1, b1, a11)


# speedup vs baseline: 1.0615x; 1.0348x over previous
import functools

import jax
import jax.numpy as jnp
from jax import lax
from jax.experimental import pallas as pl
from jax.experimental.pallas import tpu as pltpu
from jax.experimental.pallas import tpu_sc as plsc

N_NODES = 100000
FT = 128
NB_REL = 4
DEG = 32
BATCH = 4096
SEGS = BATCH * NB_REL

NC = 2
NS = 16
NW = NC * NS
SEG_PER_W = SEGS // NW
SEGS_PER_CHUNK = NB_REL
ROWS_PER_CHUNK = SEGS_PER_CHUNK * DEG
CHUNKS = SEG_PER_W // SEGS_PER_CHUNK
ROWS_W = SEG_PER_W // NB_REL
LANES = 16
VPR = FT // LANES
NBUF = 4


def _seg_mean_body(idx_hbm, feat_hbm, out_hbm, idx_v, idx_f, rows_v, out_v,
                   gs0, gs1, gs2, gs3, osem):
    gsems = (gs0, gs1, gs2, gs3)
    wid = lax.axis_index("s") * NC + lax.axis_index("c")
    seg_base = wid * SEG_PER_W

    half = ROWS_W // 2
    pltpu.sync_copy(idx_hbm.at[pl.ds(wid * ROWS_W, half)], idx_v)

    def flatten(c, fb):
        cc = lax.rem(c, half)
        for r in range(NB_REL):
            for h in range(DEG // LANES):
                idx_f[fb, pl.ds(r * DEG + h * LANES, LANES)] = (
                    idx_v[cc, r, pl.ds(h * LANES, LANES)])

    def gather(c, b):
        return pltpu.make_async_copy(feat_hbm.at[idx_f.at[b]], rows_v.at[b],
                                     gsems[b])

    def out_copy(c):
        dst = out_hbm.at[pl.ds(seg_base + c * SEGS_PER_CHUNK,
                               SEGS_PER_CHUNK)]
        return pltpu.make_async_copy(out_v, dst, osem)

    for b in range(NBUF - 1):
        flatten(b, b)
        gather(b, b).start()

    @pl.loop(0, CHUNKS, step=NBUF)
    def _group(c0):
        for b in range(NBUF):
            c = c0 + b
            gather(c, b).wait()
            nxt = c + NBUF - 1
            nb = (b + NBUF - 1) % NBUF

            @pl.when(nxt == CHUNKS // 2)
            def _():
                pltpu.sync_copy(
                    idx_hbm.at[pl.ds(wid * ROWS_W + half, half)], idx_v)

            @pl.when(nxt < CHUNKS)
            def _():
                flatten(nxt, nb)
                gather(nxt, nb).start()

            @pl.when(c >= 1)
            def _():
                out_copy(c - 1).wait()

            for s in range(SEGS_PER_CHUNK):
                def body(d, acc):
                    return tuple(
                        acc[l] + rows_v[b, s * DEG + d,
                                        pl.ds(l * LANES, LANES)]
                        for l in range(VPR)
                    )
                acc = lax.fori_loop(
                    0, DEG, body,
                    tuple(jnp.zeros((LANES,), jnp.float32)
                          for _ in range(VPR)))
                for l in range(VPR):
                    out_v[s, pl.ds(l * LANES, LANES)] = acc[l] * (1.0 / DEG)

            out_copy(c).start()

    out_copy(CHUNKS - 1).wait()


_seg_mean = functools.partial(
    pl.kernel,
    out_type=jax.ShapeDtypeStruct((SEGS, FT), jnp.float32),
    mesh=plsc.VectorSubcoreMesh(
        core_axis_name="c", subcore_axis_name="s",
        num_cores=NC, num_subcores=NS),
    compiler_params=pltpu.CompilerParams(
        skip_device_barrier=True,
        disable_bounds_checks=True,
        disable_semaphore_checks=True,
    ),
    scratch_types=[
        pltpu.VMEM((ROWS_W // 2, NB_REL, DEG), jnp.int32),
        pltpu.VMEM((NBUF, NB_REL * DEG), jnp.int32),
        pltpu.VMEM((NBUF, ROWS_PER_CHUNK, FT), jnp.float32),
        pltpu.VMEM((SEGS_PER_CHUNK, FT), jnp.float32),
    ] + [pltpu.SemaphoreType.DMA] * (NBUF + 1),
)(_seg_mean_body)


def _gcn_body(x_ref, w_ref, b_ref, a_ref, o_ref):
    a = a_ref[0, 0]
    x = x_ref[...]
    acc = jnp.zeros((x.shape[0], FT), jnp.float32)
    for r in range(NB_REL):
        h = jnp.dot(x[:, r, :], w_ref[r], preferred_element_type=jnp.float32)
        h = h + b_ref[r][None, :]
        h = jnp.where(h > 0, h, a * h)
        acc = acc + h
    o_ref[...] = acc * (1.0 / NB_REL)


def _gcn(v_in, W1, b1, a11):
    BB = 1024
    return pl.pallas_call(
        _gcn_body,
        grid=(BATCH // BB,),
        in_specs=[
            pl.BlockSpec((BB, NB_REL, FT), lambda i: (i, 0, 0)),
            pl.BlockSpec((NB_REL, FT, FT), lambda i: (0, 0, 0)),
            pl.BlockSpec((NB_REL, FT), lambda i: (0, 0)),
            pl.BlockSpec(memory_space=pltpu.SMEM),
        ],
        out_specs=pl.BlockSpec((BB, FT), lambda i: (i, 0)),
        out_shape=jax.ShapeDtypeStruct((BATCH, FT), jnp.float32),
    )(v_in, W1, b1, a11)


def kernel(node_list, neighbor_idx, features, k, W1, b1, prelu_a):
    del node_list, k
    idx3 = neighbor_idx.astype(jnp.int32)
    v_in = _seg_mean(idx3, features)
    v_in = v_in.reshape(BATCH, NB_REL, FT)
    a11 = jnp.asarray(prelu_a, jnp.float32).reshape(1, 1)
    return _gcn(v_in, W1, b1, a11)
